# baseline (speedup 1.0000x reference)
import functools

import jax
import jax.numpy as jnp
from jax import lax
from jax.experimental import pallas as pl
from jax.experimental.pallas import tpu as pltpu
from jax.experimental.pallas import tpu_sc as plsc

NUM_EXPERTS = 32
HIDDEN = 2880
TOP_K = 4
N_TOKENS = 8192
N_WORKERS = 32
TPW = N_TOKENS // N_WORKERS
GROUPS = TPW // 16


def _mm_body(x_ref, w_ref, b_ref, out_ref):
    x = x_ref[...]
    w = w_ref[...]
    lg = jax.lax.dot_general(
        w, x, (((1,), (1,)), ((), ())), preferred_element_type=jnp.float32)
    out_ref[0] = lg + b_ref[...]


def _tc_logits(hidden_states, weight, bias):
    b2 = bias.reshape(NUM_EXPERTS, 1)
    return pl.pallas_call(
        _mm_body,
        grid=(N_WORKERS,),
        in_specs=[
            pl.BlockSpec((TPW, HIDDEN), lambda i: (i, 0)),
            pl.BlockSpec((NUM_EXPERTS, HIDDEN), lambda i: (0, 0)),
            pl.BlockSpec((NUM_EXPERTS, 1), lambda i: (0, 0)),
        ],
        out_specs=pl.BlockSpec((1, NUM_EXPERTS, TPW), lambda i: (i, 0, 0)),
        out_shape=jax.ShapeDtypeStruct((N_WORKERS, NUM_EXPERTS, TPW),
                                       jnp.float32),
    )(hidden_states, weight, b2)


@functools.partial(
    pl.kernel,
    mesh=plsc.VectorSubcoreMesh(core_axis_name="c", subcore_axis_name="s"),
    out_type=[
        jax.ShapeDtypeStruct((N_TOKENS, NUM_EXPERTS), jnp.float32),
        jax.ShapeDtypeStruct((N_TOKENS, TOP_K), jnp.int32),
    ],
    scratch_types=[
        pltpu.VMEM((NUM_EXPERTS * TPW,), jnp.float32),
        pltpu.VMEM((TPW, NUM_EXPERTS), jnp.float32),
        pltpu.VMEM((TPW, TOP_K), jnp.int32),
    ],
    compiler_params=pltpu.CompilerParams(needs_layout_passes=False),
)
def _sc_route(l3_hbm, scores_hbm, sel_hbm, lbuf, scorebuf, selbuf):
    wid = lax.axis_index("s") * 2 + lax.axis_index("c")
    base = wid * TPW
    pltpu.sync_copy(l3_hbm.at[pl.ds(base * NUM_EXPERTS, TPW * NUM_EXPERTS)],
                    lbuf)
    iota16 = lax.iota(jnp.int32, 16)
    zero16f = jnp.zeros((16,), jnp.float32)

    def zero_body(i, carry):
        scorebuf[i, pl.ds(0, 16)] = zero16f
        scorebuf[i, pl.ds(16, 16)] = zero16f
        return carry

    lax.fori_loop(0, TPW, zero_body, 0)

    def group_body(g, carry):
        row0 = g * 16
        neg = jnp.full((16,), -jnp.inf, jnp.float32)
        zi = jnp.zeros((16,), jnp.int32)
        v0 = lbuf[pl.ds(row0, 16)]
        i0 = zi
        v1, v2, v3 = neg, neg, neg
        i1, i2, i3 = zi, zi, zi
        for e in range(1, NUM_EXPERTS):
            x = lbuf[pl.ds(e * TPW + row0, 16)]
            ei = jnp.full((16,), e, jnp.int32)
            m0 = x > v0
            m1 = x > v1
            m2 = x > v2
            m3 = x > v3
            v0, v1, v2, v3, i0, i1, i2, i3 = (
                jnp.where(m0, x, v0),
                jnp.where(m0, v0, jnp.where(m1, x, v1)),
                jnp.where(m1, v1, jnp.where(m2, x, v2)),
                jnp.where(m2, v2, jnp.where(m3, x, v3)),
                jnp.where(m0, ei, i0),
                jnp.where(m0, i0, jnp.where(m1, ei, i1)),
                jnp.where(m1, i1, jnp.where(m2, ei, i2)),
                jnp.where(m2, i2, jnp.where(m3, ei, i3)),
            )
        e1 = jnp.exp(v1 - v0)
        e2 = jnp.exp(v2 - v0)
        e3 = jnp.exp(v3 - v0)
        r = 1.0 / (1.0 + e1 + e2 + e3)
        rows = row0 + iota16
        zi16 = jnp.zeros((16,), jnp.int32)
        plsc.store_scatter(scorebuf, [rows, i0], r)
        plsc.store_scatter(scorebuf, [rows, i1], e1 * r)
        plsc.store_scatter(scorebuf, [rows, i2], e2 * r)
        plsc.store_scatter(scorebuf, [rows, i3], e3 * r)
        plsc.store_scatter(selbuf, [rows, zi16], i0)
        plsc.store_scatter(selbuf, [rows, zi16 + 1], i1)
        plsc.store_scatter(selbuf, [rows, zi16 + 2], i2)
        plsc.store_scatter(selbuf, [rows, zi16 + 3], i3)
        return carry

    lax.fori_loop(0, GROUPS, group_body, 0)
    pltpu.sync_copy(scorebuf, scores_hbm.at[pl.ds(base, TPW)])
    pltpu.sync_copy(selbuf, sel_hbm.at[pl.ds(base, TPW)])


@jax.jit
def kernel(hidden_states, weight, bias):
    l3 = _tc_logits(hidden_states, weight, bias)
    scores, sel_idx = _sc_route(l3.reshape(-1))
    return scores, sel_idx


# speedup vs baseline: 1.0893x; 1.0893x over previous
import jax
import jax.numpy as jnp
from jax.experimental import pallas as pl

NUM_EXPERTS = 32
HIDDEN = 2880
TOP_K = 4
BLOCK_T = 512


def _route(logits):
    iota = jax.lax.broadcasted_iota(jnp.int32, logits.shape, 1)
    cur = logits
    vals, idxs = [], []
    for _ in range(TOP_K):
        m = jnp.max(cur, axis=1, keepdims=True)
        amax = jnp.min(jnp.where(cur == m, iota, NUM_EXPERTS),
                       axis=1, keepdims=True)
        vals.append(m)
        idxs.append(amax)
        cur = jnp.where(iota == amax, -jnp.inf, cur)
    v = jnp.concatenate(vals, axis=1)
    e = jnp.exp(v - vals[0])
    p = e / jnp.sum(e, axis=1, keepdims=True)
    scores = jnp.zeros_like(logits)
    for k in range(TOP_K):
        scores = jnp.where(iota == idxs[k], p[:, k:k + 1], scores)
    return scores, jnp.concatenate(idxs, axis=1).astype(jnp.int32)


def _router_body(x1_ref, x2_ref, w_ref, b_ref, scores_ref, idx_ref):
    w = w_ref[...]
    b = b_ref[...]
    for half, x_ref in enumerate((x1_ref, x2_ref)):
        logits = jax.lax.dot_general(
            x_ref[...], w, (((1,), (1,)), ((), ())),
            preferred_element_type=jnp.float32) + b
        scores, idx = _route(logits)
        sl = pl.ds(half * BLOCK_T, BLOCK_T)
        scores_ref[sl, :] = scores
        idx_ref[sl, :] = idx


@jax.jit
def kernel(hidden_states, weight, bias):
    n_tokens = hidden_states.shape[0]
    grid = (n_tokens // (2 * BLOCK_T),)
    b2d = bias.reshape(1, NUM_EXPERTS)
    scores, sel_idx = pl.pallas_call(
        _router_body,
        grid=grid,
        in_specs=[
            pl.BlockSpec((BLOCK_T, HIDDEN), lambda i: (2 * i, 0)),
            pl.BlockSpec((BLOCK_T, HIDDEN), lambda i: (2 * i + 1, 0)),
            pl.BlockSpec((NUM_EXPERTS, HIDDEN), lambda i: (0, 0)),
            pl.BlockSpec((1, NUM_EXPERTS), lambda i: (0, 0)),
        ],
        out_specs=[
            pl.BlockSpec((2 * BLOCK_T, NUM_EXPERTS), lambda i: (i, 0)),
            pl.BlockSpec((2 * BLOCK_T, TOP_K), lambda i: (i, 0)),
        ],
        out_shape=[
            jax.ShapeDtypeStruct((n_tokens, NUM_EXPERTS), jnp.float32),
            jax.ShapeDtypeStruct((n_tokens, TOP_K), jnp.int32),
        ],
    )(hidden_states, hidden_states, weight, b2d)
    return scores, sel_idx
